# Initial kernel scaffold; baseline (speedup 1.0000x reference)
import jax
import jax.numpy as jnp
from jax.experimental import pallas as pl


def kernel(x, table):
    raise NotImplementedError("write your pallas kernel here")


# speedup vs baseline: 8.3741x; 8.3741x over previous
import math
import functools

import jax
import jax.numpy as jnp
from jax import lax
from jax.experimental import pallas as pl
from jax.experimental.pallas import tpu as pltpu
from jax.experimental.pallas import tpu_sc as plsc

VOCAB = 100001
D = 128
SCALE = math.sqrt(128.0)

NC = 2
NS = 16
NW = NC * NS

B = 4096 * 200
BPW = B // NW
C = 128
NCH = BPW // C
NBUF = 5
K = 2
NG = NCH // NBUF


def _scale_body(t_ref, o_ref):
    o_ref[...] = t_ref[...] * SCALE


def _scaled_table(table):
    rb = 8192
    return pl.pallas_call(
        _scale_body,
        grid=(pl.cdiv(VOCAB, rb),),
        in_specs=[pl.BlockSpec((rb, D), lambda i: (i, 0))],
        out_specs=pl.BlockSpec((rb, D), lambda i: (i, 0)),
        out_shape=jax.ShapeDtypeStruct((VOCAB, D), jnp.float32),
    )(table)


def _gather_kernel(x_hbm, table_hbm, out_hbm, idx_v, *bufs_and_sems):
    rows = bufs_and_sems[:NBUF]
    gsem = bufs_and_sems[NBUF:2 * NBUF]
    ssem = bufs_and_sems[2 * NBUF:3 * NBUF]

    wid = lax.axis_index("s") * NC + lax.axis_index("c")
    base = wid * BPW

    pltpu.sync_copy(x_hbm.at[wid], idx_v)

    def fire_gather(j, b):
        pltpu.async_copy(table_hbm.at[idx_v.at[j]], rows[b], gsem[b])

    def wait_gather(j, b):
        pltpu.make_async_copy(table_hbm.at[idx_v.at[j]], rows[b], gsem[b]).wait()

    def fire_store(j, b):
        pltpu.async_copy(rows[b], out_hbm.at[pl.ds(base + j * C, C)], ssem[b])

    def wait_store(j, b):
        pltpu.make_async_copy(
            rows[b], out_hbm.at[pl.ds(base + j * C, C)], ssem[b]).wait()

    for b in range(K):
        fire_gather(b, b)

    for b in range(NBUF):
        j = b
        wait_gather(j, b)
        fire_store(j, b)
        jn = j + K
        bn = (b + K) % NBUF
        if jn >= NBUF:
            wait_store(jn - NBUF, bn)
        fire_gather(jn, bn)

    def group(g, _):
        j0 = g * NBUF
        for b in range(NBUF):
            j = j0 + b
            wait_gather(j, b)
            fire_store(j, b)
            bn = (b + K) % NBUF
            wait_store(j + K - NBUF, bn)
            fire_gather(j + K, bn)
        return _

    lax.fori_loop(1, NG - 1, group, None)

    j0 = (NG - 1) * NBUF
    for b in range(NBUF):
        j = j0 + b
        wait_gather(j, b)
        fire_store(j, b)
        jn = j + K
        if jn < NCH:
            bn = (b + K) % NBUF
            wait_store(jn - NBUF, bn)
            fire_gather(jn, bn)

    for b in range(NBUF):
        wait_store(NCH - NBUF + b, b)


@functools.partial(
    pl.kernel,
    out_type=jax.ShapeDtypeStruct((B, D), jnp.float32),
    mesh=plsc.VectorSubcoreMesh(core_axis_name="c", subcore_axis_name="s"),
    scratch_types=(
        [pltpu.VMEM((NCH, C), jnp.int32)]
        + [pltpu.VMEM((C, D), jnp.float32) for _ in range(NBUF)]
        + [pltpu.SemaphoreType.DMA for _ in range(2 * NBUF)]
    ),
)
def _sc_gather(x_hbm, table_hbm, out_hbm, idx_v, *bufs_and_sems):
    _gather_kernel(x_hbm, table_hbm, out_hbm, idx_v, *bufs_and_sems)


def kernel(x, table):
    scaled = _scaled_table(table)
    xw = x.reshape(NW, NCH, C).astype(jnp.int32)
    out = _sc_gather(xw, scaled)
    return out.reshape(4096, 200, D)


# speedup vs baseline: 9.2466x; 1.1042x over previous
import math
import functools

import jax
import jax.numpy as jnp
from jax import lax
from jax.experimental import pallas as pl
from jax.experimental.pallas import tpu as pltpu
from jax.experimental.pallas import tpu_sc as plsc

VOCAB = 100001
D = 128
SCALE = math.sqrt(128.0)

NC = 2
NS = 16
NW = NC * NS

B = 4096 * 200
BPW = B // NW
C = 128
NCH = BPW // C
NBUF = 5
K = 2
NG = NCH // NBUF
UNROLL = 8


def _gather_kernel(x_hbm, table_hbm, out_hbm, idx_v, *bufs_and_sems):
    rows = bufs_and_sems[:NBUF]
    gsem = bufs_and_sems[NBUF:2 * NBUF]
    ssem = bufs_and_sems[2 * NBUF:3 * NBUF]

    wid = lax.axis_index("s") * NC + lax.axis_index("c")
    base = wid * BPW

    pltpu.sync_copy(x_hbm.at[wid], idx_v)

    def fire_gather(j, b):
        pltpu.async_copy(table_hbm.at[idx_v.at[j]], rows[b], gsem[b])

    def wait_gather(j, b):
        pltpu.make_async_copy(table_hbm.at[idx_v.at[j]], rows[b], gsem[b]).wait()

    def fire_store(j, b):
        pltpu.async_copy(rows[b], out_hbm.at[pl.ds(base + j * C, C)], ssem[b])

    def wait_store(j, b):
        pltpu.make_async_copy(
            rows[b], out_hbm.at[pl.ds(base + j * C, C)], ssem[b]).wait()

    def scale_rows(b):
        buf = rows[b]

        def row(r, _):
            for k in range(UNROLL):
                sl = pl.ds(k * 16, 16)
                buf[r, sl] = buf[r, sl] * SCALE
            return _

        lax.fori_loop(0, C, row, None)

    for b in range(K):
        fire_gather(b, b)

    for b in range(NBUF):
        j = b
        wait_gather(j, b)
        scale_rows(b)
        fire_store(j, b)
        jn = j + K
        bn = (b + K) % NBUF
        if jn >= NBUF:
            wait_store(jn - NBUF, bn)
        fire_gather(jn, bn)

    def group(g, _):
        j0 = g * NBUF
        for b in range(NBUF):
            j = j0 + b
            wait_gather(j, b)
            scale_rows(b)
            fire_store(j, b)
            bn = (b + K) % NBUF
            wait_store(j + K - NBUF, bn)
            fire_gather(j + K, bn)
        return _

    lax.fori_loop(1, NG - 1, group, None)

    j0 = (NG - 1) * NBUF
    for b in range(NBUF):
        j = j0 + b
        wait_gather(j, b)
        scale_rows(b)
        fire_store(j, b)
        jn = j + K
        if jn < NCH:
            bn = (b + K) % NBUF
            wait_store(jn - NBUF, bn)
            fire_gather(jn, bn)

    for b in range(NBUF):
        wait_store(NCH - NBUF + b, b)


@functools.partial(
    pl.kernel,
    out_type=jax.ShapeDtypeStruct((B, D), jnp.float32),
    mesh=plsc.VectorSubcoreMesh(core_axis_name="c", subcore_axis_name="s"),
    scratch_types=(
        [pltpu.VMEM((NCH, C), jnp.int32)]
        + [pltpu.VMEM((C, D), jnp.float32) for _ in range(NBUF)]
        + [pltpu.SemaphoreType.DMA for _ in range(2 * NBUF)]
    ),
)
def _sc_gather(x_hbm, table_hbm, out_hbm, idx_v, *bufs_and_sems):
    _gather_kernel(x_hbm, table_hbm, out_hbm, idx_v, *bufs_and_sems)


def kernel(x, table):
    xw = x.reshape(NW, NCH, C).astype(jnp.int32)
    out = _sc_gather(xw, table)
    return out.reshape(4096, 200, D)
